# baseline (speedup 1.0000x reference)
import jax
import jax.numpy as jnp
from jax import lax
from jax.experimental import pallas as pl
from jax.experimental.pallas import tpu as pltpu
from jax.experimental.pallas import tpu_sc as plsc

VOCAB = 1000
EMB_DIM = 4
B, L = 4096, 50

NC, NS = 2, 16
NW = NC * NS
VT = VOCAB // 8
NVT = 4
NBT = B // 128
HB = NBT // 2
LANES = 16


def _table_body(emb_ref, w_ref, b_ref, out_ref):
    acc = jax.lax.dot_general(
        emb_ref[...], w_ref[...],
        (((1,), (1,)), ((), ())),
        preferred_element_type=jnp.float32,
    )
    out_ref[...] = acc + b_ref[...]


def _make_table(emb_table, W, b):
    return pl.pallas_call(
        _table_body,
        out_shape=jax.ShapeDtypeStruct((VOCAB, VOCAB), jnp.float32),
    )(emb_table, W, b.reshape(1, VOCAB))


def _sc_gather_body(table_hbm, xt_hbm, out_hbm,
                    tloc, ib0, ib1, ob0, ob1,
                    is0, is1, wo0, wo1):
    c = lax.axis_index("c")
    s = lax.axis_index("s")
    wid = s * NC + c
    vt0 = jnp.minimum(wid * NVT, VT - NVT)

    pltpu.sync_copy(
        table_hbm.at[pl.ds(0, VOCAB), pl.ds(vt0 * 8, NVT * 8)], tloc)

    def idx_dma(l, ib, sem):
        return pltpu.make_async_copy(xt_hbm.at[l], ib, sem)

    def out_dma(l, vt_k, h, ob, sem):
        dst = out_hbm.at[l, vt0 + vt_k, pl.ds(h * HB, HB)]
        return pltpu.make_async_copy(ob, dst, sem)

    def fill(ib, ob, colvs, h):
        @plsc.parallel_loop(0, HB, unroll=2)
        def _(btl):
            for bi16 in range(8):
                off = (h * HB) * 128 + btl * 128 + bi16 * LANES
                idxvec = ib[pl.ds(off, LANES)]
                for vi in range(8):
                    ob[btl, vi, pl.ds(bi16 * LANES, LANES)] = (
                        plsc.load_gather(tloc, [idxvec, colvs[vi]]))

    idx_dma(0, ib0, is0).start()
    idx_dma(1, ib1, is1).start()

    def lpair(t, carry):
        for j in range(2):
            l = t * 2 + j
            ib, isem = (ib0, is0) if j == 0 else (ib1, is1)
            idx_dma(l, ib, isem).wait()

            def vt_body(vt_k, carry2):
                colvs = [jnp.full((LANES,), vt_k * 8 + vi, jnp.int32)
                         for vi in range(8)]
                for h in range(2):
                    ob, wsem = (ob0, wo0) if h == 0 else (ob1, wo1)

                    @pl.when(l * NVT + vt_k >= 1)
                    def _():
                        out_dma(l, vt_k, h, ob, wsem).wait()

                    fill(ib, ob, colvs, h)
                    out_dma(l, vt_k, h, ob, wsem).start()
                return carry2

            lax.fori_loop(0, NVT, vt_body, 0)

            @pl.when(l + 2 < L)
            def _():
                idx_dma(l + 2, ib, isem).start()
        return carry

    lax.fori_loop(0, L // 2, lpair, 0)
    out_dma(L - 1, NVT - 1, 0, ob0, wo0).wait()
    out_dma(L - 1, NVT - 1, 1, ob1, wo1).wait()


def _gather_rows(table, x_t):
    mesh = plsc.VectorSubcoreMesh(core_axis_name="c", subcore_axis_name="s")
    return pl.kernel(
        _sc_gather_body,
        out_type=jax.ShapeDtypeStruct((L, VT, NBT, 8, 128), jnp.float32),
        mesh=mesh,
        scratch_types=[
            pltpu.VMEM((VOCAB, NVT * 8), jnp.float32),
            pltpu.VMEM((B,), jnp.int32),
            pltpu.VMEM((B,), jnp.int32),
            pltpu.VMEM((HB, 8, 128), jnp.float32),
            pltpu.VMEM((HB, 8, 128), jnp.float32),
            pltpu.SemaphoreType.DMA,
            pltpu.SemaphoreType.DMA,
            pltpu.SemaphoreType.DMA,
            pltpu.SemaphoreType.DMA,
        ],
        compiler_params=pltpu.CompilerParams(
            use_tc_tiling_on_sc=False, needs_layout_passes=False
        ),
    )(table, x_t)


def kernel(x, emb_table, W, b):
    table = _make_table(emb_table, W, b)
    x_t = x.astype(jnp.int32).T
    out5 = _gather_rows(table, x_t)
    return out5.transpose(2, 4, 0, 1, 3).reshape(B, L, VOCAB)


# speedup vs baseline: 2.2258x; 2.2258x over previous
import jax
import jax.numpy as jnp
from jax import lax
from jax.experimental import pallas as pl
from jax.experimental.pallas import tpu as pltpu
from jax.experimental.pallas import tpu_sc as plsc

VOCAB = 1000
EMB_DIM = 4
B, L = 4096, 50

NC, NS = 2, 16
NW = NC * NS
VT = VOCAB // 8
NVT = 4
NBT = B // 128
HB = NBT // 2
LANES = 16


def _table_body(emb_ref, w_ref, b_ref, out_ref):
    acc = jax.lax.dot_general(
        emb_ref[...], w_ref[...],
        (((1,), (1,)), ((), ())),
        preferred_element_type=jnp.float32,
    )
    out_ref[...] = acc + b_ref[...]


def _make_table(emb_table, W, b):
    return pl.pallas_call(
        _table_body,
        out_shape=jax.ShapeDtypeStruct((VOCAB, VOCAB), jnp.float32),
    )(emb_table, W, b.reshape(1, VOCAB))


def _sc_gather_body(table_hbm, xt_hbm, out_hbm,
                    tloc, ib0, ib1, ob0, ob1,
                    is0, is1, wo0, wo1):
    c = lax.axis_index("c")
    s = lax.axis_index("s")
    wid = s * NC + c
    vt0 = jnp.minimum(wid * NVT, VT - NVT)

    pltpu.sync_copy(
        table_hbm.at[pl.ds(0, VOCAB), pl.ds(vt0 * 8, NVT * 8)],
        tloc.at[pl.ds(0, VOCAB), pl.ds(0, NVT * 8)])

    def idx_dma(l, ib, sem):
        return pltpu.make_async_copy(xt_hbm.at[l], ib, sem)

    def out_dma(l, vt_k, h, ob, sem):
        dst = out_hbm.at[l, vt0 + vt_k, pl.ds(h * HB, HB)]
        return pltpu.make_async_copy(ob, dst, sem)

    def fill(ib, ob, colvs, h):
        @plsc.parallel_loop(0, HB, unroll=2)
        def _(btl):
            for bi16 in range(8):
                off = (h * HB) * 128 + btl * 128 + bi16 * LANES
                idxvec = ib[pl.ds(off, LANES)]
                for vi in range(8):
                    ob[btl, vi, pl.ds(bi16 * LANES, LANES)] = (
                        plsc.load_gather(tloc, [idxvec, colvs[vi]]))

    idx_dma(0, ib0, is0).start()
    idx_dma(1, ib1, is1).start()

    def lpair(t, carry):
        for j in range(2):
            l = t * 2 + j
            ib, isem = (ib0, is0) if j == 0 else (ib1, is1)
            idx_dma(l, ib, isem).wait()

            def vt_body(vt_k, carry2):
                colvs = [jnp.full((LANES,), vt_k * 8 + vi, jnp.int32)
                         for vi in range(8)]
                for h in range(2):
                    ob, wsem = (ob0, wo0) if h == 0 else (ob1, wo1)

                    @pl.when(l * NVT + vt_k >= 1)
                    def _():
                        out_dma(l, vt_k, h, ob, wsem).wait()

                    fill(ib, ob, colvs, h)
                    out_dma(l, vt_k, h, ob, wsem).start()
                return carry2

            lax.fori_loop(0, NVT, vt_body, 0)

            @pl.when(l + 2 < L)
            def _():
                idx_dma(l + 2, ib, isem).start()
        return carry

    lax.fori_loop(0, L // 2, lpair, 0)
    out_dma(L - 1, NVT - 1, 0, ob0, wo0).wait()
    out_dma(L - 1, NVT - 1, 1, ob1, wo1).wait()


def _gather_rows(table, x_t):
    mesh = plsc.VectorSubcoreMesh(core_axis_name="c", subcore_axis_name="s")
    return pl.kernel(
        _sc_gather_body,
        out_type=jax.ShapeDtypeStruct((L, VT, NBT, 8, 128), jnp.float32),
        mesh=mesh,
        scratch_types=[
            pltpu.VMEM((VOCAB, 40), jnp.float32),
            pltpu.VMEM((B,), jnp.int32),
            pltpu.VMEM((B,), jnp.int32),
            pltpu.VMEM((HB, 8, 128), jnp.float32),
            pltpu.VMEM((HB, 8, 128), jnp.float32),
            pltpu.SemaphoreType.DMA,
            pltpu.SemaphoreType.DMA,
            pltpu.SemaphoreType.DMA,
            pltpu.SemaphoreType.DMA,
        ],
        compiler_params=pltpu.CompilerParams(
            use_tc_tiling_on_sc=False, needs_layout_passes=False
        ),
    )(table, x_t)


def kernel(x, emb_table, W, b):
    table = _make_table(emb_table, W, b)
    x_t = x.astype(jnp.int32).T
    out5 = _gather_rows(table, x_t)
    return out5.transpose(2, 4, 0, 1, 3).reshape(B, L, VOCAB)


# speedup vs baseline: 3.4928x; 1.5692x over previous
import jax
import jax.numpy as jnp
from jax import lax
from jax.experimental import pallas as pl
from jax.experimental.pallas import tpu as pltpu
from jax.experimental.pallas import tpu_sc as plsc

VOCAB = 1000
EMB_DIM = 4
B, L = 4096, 50

NC, NS = 2, 16
NW = NC * NS
BT = B // NW
VT = VOCAB // 8
QW = 16
NQ = BT // QW
LANES = 16


def _table_body(emb_ref, w_ref, b_ref, out_ref):
    acc = jax.lax.dot_general(
        emb_ref[...], w_ref[...],
        (((1,), (1,)), ((), ())),
        preferred_element_type=jnp.float32,
    )
    out_ref[...] = acc + b_ref[...]


def _make_table(emb_table, W, b):
    return pl.pallas_call(
        _table_body,
        out_shape=jax.ShapeDtypeStruct((VOCAB, VOCAB), jnp.float32),
    )(emb_table, W, b.reshape(1, VOCAB))


def _sc_gather_body(table_hbm, xt_hbm, out_hbm,
                    idx_v, rows0, rows1, rows2, tr0, tr1, tr2,
                    gs0, gs1, gs2, ws0, ws1, ws2):
    c = lax.axis_index("c")
    s = lax.axis_index("s")
    wid = s * NC + c

    iota = lax.iota(jnp.int32, LANES)
    bvecs = [iota + b2 for b2 in range(0, QW, LANES)]

    pltpu.sync_copy(xt_hbm.at[pl.ds(0, L), pl.ds(wid * BT, BT)], idx_v)

    NQTOT = L * NQ

    def gather(Q, rows_v, sem):
        l = Q // NQ
        q = lax.rem(Q, NQ)
        src = table_hbm.at[idx_v.at[l, pl.ds(q * QW, QW)]]
        return pltpu.make_async_copy(src, rows_v, sem)

    def out_dma(Q, tr_v, sem):
        l = Q // NQ
        q = lax.rem(Q, NQ)
        dst = out_hbm.at[l, pl.ds(0, VT), wid, pl.ds(0, 8), pl.ds(q * QW, QW)]
        return pltpu.make_async_copy(tr_v, dst, sem)

    def transpose(rows_v, tr_v):
        @plsc.parallel_loop(0, VT, unroll=4)
        def _(vt):
            for vi in range(8):
                col_v = jnp.full((LANES,), vt * 8 + vi, jnp.int32)
                for k, bv in enumerate(bvecs):
                    tr_v[vt, vi, pl.ds(k * LANES, LANES)] = (
                        plsc.load_gather(rows_v, [bv, col_v]))

    bufs = [(rows0, gs0, tr0, ws0), (rows1, gs1, tr1, ws1),
            (rows2, gs2, tr2, ws2)]
    NDEEP = len(bufs)

    for k in range(NDEEP):
        gather(k, bufs[k][0], bufs[k][1]).start()

    NTRIPLE = -(-NQTOT // NDEEP)

    def triple(t3, carry):
        for k in range(NDEEP):
            Q = t3 * NDEEP + k
            rows_v, gsem, tr_v, wsem = bufs[k]

            @pl.when(Q < NQTOT)
            def _():
                gather(Q, rows_v, gsem).wait()

                @pl.when(Q >= NDEEP)
                def _():
                    out_dma(Q, tr_v, wsem).wait()

                transpose(rows_v, tr_v)
                out_dma(Q, tr_v, wsem).start()

                @pl.when(Q + NDEEP < NQTOT)
                def _():
                    gather(Q + NDEEP, rows_v, gsem).start()
        return carry

    lax.fori_loop(0, NTRIPLE, triple, 0)
    for Q in range(NQTOT - NDEEP, NQTOT):
        _, _, tr_v, wsem = bufs[Q % NDEEP]
        out_dma(Q, tr_v, wsem).wait()


def _gather_rows(table, x_t):
    mesh = plsc.VectorSubcoreMesh(core_axis_name="c", subcore_axis_name="s")
    return pl.kernel(
        _sc_gather_body,
        out_type=jax.ShapeDtypeStruct((L, VT, NW, 8, BT), jnp.float32),
        mesh=mesh,
        scratch_types=[
            pltpu.VMEM((L, BT), jnp.int32),
            pltpu.VMEM((QW, VOCAB), jnp.float32),
            pltpu.VMEM((QW, VOCAB), jnp.float32),
            pltpu.VMEM((QW, VOCAB), jnp.float32),
            pltpu.VMEM((VT, 8, QW), jnp.float32),
            pltpu.VMEM((VT, 8, QW), jnp.float32),
            pltpu.VMEM((VT, 8, QW), jnp.float32),
            pltpu.SemaphoreType.DMA,
            pltpu.SemaphoreType.DMA,
            pltpu.SemaphoreType.DMA,
            pltpu.SemaphoreType.DMA,
            pltpu.SemaphoreType.DMA,
            pltpu.SemaphoreType.DMA,
        ],
        compiler_params=pltpu.CompilerParams(
            use_tc_tiling_on_sc=False, needs_layout_passes=False
        ),
    )(table, x_t)


def kernel(x, emb_table, W, b):
    table = _make_table(emb_table, W, b)
    x_t = x.astype(jnp.int32).T
    out5 = _gather_rows(table, x_t)
    return out5.transpose(2, 4, 0, 1, 3).reshape(B, L, VOCAB)
